# baseline (speedup 1.0000x reference)
import math

import jax
import jax.numpy as jnp
from jax.experimental import pallas as pl
from jax.experimental.pallas import tpu as pltpu

D_MODEL = 1024
FPB = 43
BPB = 4
BPP = 4
DPS = D_MODEL // 4
BS = 512


def _add_pe_kernel(fe_ref, be_ref, ba_ref, fp_ref, x_ref, o_ref):
    j = pl.program_id(1)
    row = j * BS + jax.lax.broadcasted_iota(jnp.int32, (BS, 1), 0)
    beat_pos = row % FPB
    bar_pos = (row // FPB) % BPB
    phrase_pos = (row // (FPB * BPB)) % BPP
    cols43 = jax.lax.broadcasted_iota(jnp.int32, (BS, FPB), 1)
    cols4 = jax.lax.broadcasted_iota(jnp.int32, (BS, BPB), 1)
    oh_f = (cols43 == beat_pos).astype(jnp.float32)
    oh_b = (cols4 == bar_pos).astype(jnp.float32)
    oh_p = (cols4 == phrase_pos).astype(jnp.float32)
    f = jnp.dot(oh_f, fe_ref[...], preferred_element_type=jnp.float32)
    b = jnp.dot(oh_b, be_ref[...], preferred_element_type=jnp.float32)
    p = jnp.dot(oh_p, ba_ref[...], preferred_element_type=jnp.float32)
    freq = fp_ref[0:1, :]
    phase = fp_ref[1:2, :]
    abs_pe = jnp.sin(row.astype(jnp.float32) * freq + phase)
    enc = jnp.concatenate([f, b, p, abs_pe], axis=-1)
    o_ref[...] = x_ref[...] + enc[None, :, :]


def kernel(x, frame_embed, beat_embed, bar_embed, pe):
    B, S, D = x.shape
    lane = jnp.arange(DPS)
    freq = jnp.exp((lane // 2 * 2).astype(jnp.float32) * (-math.log(10000.0) / DPS))
    phase = jnp.where(lane % 2 == 1, jnp.float32(math.pi / 2), jnp.float32(0.0))
    fp = jnp.zeros((8, DPS), x.dtype).at[0].set(freq).at[1].set(phase)
    return pl.pallas_call(
        _add_pe_kernel,
        grid=(B, S // BS),
        in_specs=[
            pl.BlockSpec((FPB, DPS), lambda b, j: (0, 0)),
            pl.BlockSpec((BPB, DPS), lambda b, j: (0, 0)),
            pl.BlockSpec((BPP, DPS), lambda b, j: (0, 0)),
            pl.BlockSpec((8, DPS), lambda b, j: (0, 0)),
            pl.BlockSpec((1, BS, D), lambda b, j: (b, j, 0)),
        ],
        out_specs=pl.BlockSpec((1, BS, D), lambda b, j: (b, j, 0)),
        out_shape=jax.ShapeDtypeStruct((B, S, D), x.dtype),
        compiler_params=pltpu.CompilerParams(
            dimension_semantics=("parallel", "parallel"),
        ),
    )(frame_embed, beat_embed, bar_embed, fp, x)


# speedup vs baseline: 1.3043x; 1.3043x over previous
import functools
import math

import jax
import jax.numpy as jnp
from jax import lax
from jax.experimental import pallas as pl
from jax.experimental.pallas import tpu as pltpu
from jax.experimental.pallas import tpu_sc as plsc

D_MODEL = 1024
FPB = 43
BPB = 4
BPP = 4
DPS = D_MODEL // 4
PERIOD = FPB * BPB * BPP
PAT = 768
BS = PERIOD

_info = plsc.get_sparse_core_info()
NW = _info.num_cores * _info.num_subcores
RPW = PAT // NW


@functools.partial(
    pl.kernel,
    mesh=plsc.VectorSubcoreMesh(core_axis_name="c", subcore_axis_name="s"),
    out_type=[
        jax.ShapeDtypeStruct((PAT, DPS), jnp.float32),
        jax.ShapeDtypeStruct((PAT, DPS), jnp.float32),
        jax.ShapeDtypeStruct((PAT, DPS), jnp.float32),
    ],
    scratch_types=[
        pltpu.VMEM((RPW,), jnp.int32),
        pltpu.VMEM((RPW,), jnp.int32),
        pltpu.VMEM((RPW,), jnp.int32),
        pltpu.VMEM((RPW, DPS), jnp.float32),
        pltpu.VMEM((RPW, DPS), jnp.float32),
        pltpu.VMEM((RPW, DPS), jnp.float32),
        pltpu.SemaphoreType.DMA,
    ],
)
def _gather_pattern(fe_hbm, be_hbm, ba_hbm, if_hbm, ib_hbm, ip_hbm,
                    of_hbm, ob_hbm, op_hbm,
                    if_v, ib_v, ip_v, rf_v, rb_v, rp_v, sem):
    wid = lax.axis_index("s") * _info.num_cores + lax.axis_index("c")
    base = wid * RPW
    pltpu.sync_copy(if_hbm.at[pl.ds(base, RPW)], if_v)
    pltpu.sync_copy(ib_hbm.at[pl.ds(base, RPW)], ib_v)
    pltpu.sync_copy(ip_hbm.at[pl.ds(base, RPW)], ip_v)
    pltpu.async_copy(fe_hbm.at[if_v], rf_v, sem).wait()
    pltpu.async_copy(be_hbm.at[ib_v], rb_v, sem).wait()
    pltpu.async_copy(ba_hbm.at[ip_v], rp_v, sem).wait()
    pltpu.sync_copy(rf_v, of_hbm.at[pl.ds(base, RPW)])
    pltpu.sync_copy(rb_v, ob_hbm.at[pl.ds(base, RPW)])
    pltpu.sync_copy(rp_v, op_hbm.at[pl.ds(base, RPW)])


def _add_pe_kernel(pf_ref, pb_ref, pp_ref, fp_ref, x_ref, o_ref):
    j = pl.program_id(0)
    row = j * BS + jax.lax.broadcasted_iota(jnp.int32, (BS, 1), 0)
    freq = fp_ref[0:1, :]
    phase = fp_ref[1:2, :]
    abs_pe = jnp.sin(row.astype(jnp.float32) * freq + phase)
    enc = jnp.concatenate([pf_ref[...], pb_ref[...], pp_ref[...], abs_pe], axis=-1)
    o_ref[...] = x_ref[...] + enc[None, :, :]


def kernel(x, frame_embed, beat_embed, bar_embed, pe):
    B, S, D = x.shape
    r = jnp.arange(PAT, dtype=jnp.int32)
    idx_f = r % FPB
    idx_b = (r // FPB) % BPB
    idx_p = (r // (FPB * BPB)) % BPP
    pf, pb, pp = _gather_pattern(frame_embed, beat_embed, bar_embed,
                                 idx_f, idx_b, idx_p)
    lane = jnp.arange(DPS)
    freq = jnp.exp((lane // 2 * 2).astype(jnp.float32) * (-math.log(10000.0) / DPS))
    phase = jnp.where(lane % 2 == 1, jnp.float32(math.pi / 2), jnp.float32(0.0))
    fp = jnp.zeros((8, DPS), x.dtype).at[0].set(freq).at[1].set(phase)
    return pl.pallas_call(
        _add_pe_kernel,
        grid=(pl.cdiv(S, BS),),
        in_specs=[
            pl.BlockSpec((PERIOD, DPS), lambda j: (0, 0)),
            pl.BlockSpec((PERIOD, DPS), lambda j: (0, 0)),
            pl.BlockSpec((PERIOD, DPS), lambda j: (0, 0)),
            pl.BlockSpec((8, DPS), lambda j: (0, 0)),
            pl.BlockSpec((B, BS, D), lambda j: (0, j, 0)),
        ],
        out_specs=pl.BlockSpec((B, BS, D), lambda j: (0, j, 0)),
        out_shape=jax.ShapeDtypeStruct((B, S, D), x.dtype),
        compiler_params=pltpu.CompilerParams(
            dimension_semantics=("parallel",),
        ),
    )(pf, pb, pp, fp, x)


# speedup vs baseline: 1.3660x; 1.0473x over previous
import functools
import math

import jax
import jax.numpy as jnp
from jax import lax
from jax.experimental import pallas as pl
from jax.experimental.pallas import tpu as pltpu
from jax.experimental.pallas import tpu_sc as plsc

D_MODEL = 1024
FPB = 43
BPB = 4
BPP = 4
DPS = D_MODEL // 4
PERIOD = FPB * BPB * BPP
PAT = 768
BS = PERIOD

_info = plsc.get_sparse_core_info()
NW = _info.num_cores * _info.num_subcores
RPW = PAT // NW


@functools.partial(
    pl.kernel,
    mesh=plsc.VectorSubcoreMesh(core_axis_name="c", subcore_axis_name="s"),
    out_type=[
        jax.ShapeDtypeStruct((PAT, DPS), jnp.float32),
        jax.ShapeDtypeStruct((PAT, DPS), jnp.float32),
        jax.ShapeDtypeStruct((PAT, DPS), jnp.float32),
    ],
    scratch_types=[
        pltpu.VMEM((3 * RPW,), jnp.int32),
        pltpu.VMEM((3 * RPW, DPS), jnp.float32),
        pltpu.SemaphoreType.DMA,
        pltpu.SemaphoreType.DMA,
    ],
)
def _gather_pattern(tab_hbm, idx_hbm, of_hbm, ob_hbm, op_hbm,
                    idx_v, rows_v, gsem, wsem):
    wid = lax.axis_index("s") * _info.num_cores + lax.axis_index("c")
    base = wid * RPW
    pltpu.sync_copy(idx_hbm.at[wid], idx_v)
    pltpu.async_copy(tab_hbm.at[idx_v], rows_v, gsem).wait()
    c0 = pltpu.async_copy(rows_v.at[pl.ds(0, RPW)], of_hbm.at[pl.ds(base, RPW)], wsem)
    c1 = pltpu.async_copy(rows_v.at[pl.ds(RPW, RPW)], ob_hbm.at[pl.ds(base, RPW)], wsem)
    c2 = pltpu.async_copy(rows_v.at[pl.ds(2 * RPW, RPW)], op_hbm.at[pl.ds(base, RPW)], wsem)
    c0.wait()
    c1.wait()
    c2.wait()


def _add_pe_kernel(pf_ref, pb_ref, pp_ref, fp_ref, x_ref, o_ref):
    j = pl.program_id(0)
    row = j * BS + jax.lax.broadcasted_iota(jnp.int32, (BS, 1), 0)
    freq = fp_ref[0:1, :]
    phase = fp_ref[1:2, :]
    abs_pe = jnp.sin(row.astype(jnp.float32) * freq + phase)
    enc = jnp.concatenate([pf_ref[...], pb_ref[...], pp_ref[...], abs_pe], axis=-1)
    o_ref[...] = x_ref[...] + enc[None, :, :]


def kernel(x, frame_embed, beat_embed, bar_embed, pe):
    B, S, D = x.shape
    table = jnp.concatenate([frame_embed, beat_embed, bar_embed], axis=0)
    r = jnp.arange(PAT, dtype=jnp.int32)
    idx_f = r % FPB
    idx_b = FPB + (r // FPB) % BPB
    idx_p = FPB + BPB + (r // (FPB * BPB)) % BPP
    idx_all = jnp.stack(
        [idx_f.reshape(NW, RPW), idx_b.reshape(NW, RPW), idx_p.reshape(NW, RPW)],
        axis=1,
    ).reshape(NW, 3 * RPW)
    pf, pb, pp = _gather_pattern(table, idx_all)
    lane = jnp.arange(DPS)
    freq = jnp.exp((lane // 2 * 2).astype(jnp.float32) * (-math.log(10000.0) / DPS))
    phase = jnp.where(lane % 2 == 1, jnp.float32(math.pi / 2), jnp.float32(0.0))
    fp = jnp.zeros((8, DPS), x.dtype).at[0].set(freq).at[1].set(phase)
    return pl.pallas_call(
        _add_pe_kernel,
        grid=(pl.cdiv(S, BS),),
        in_specs=[
            pl.BlockSpec((PERIOD, DPS), lambda j: (0, 0)),
            pl.BlockSpec((PERIOD, DPS), lambda j: (0, 0)),
            pl.BlockSpec((PERIOD, DPS), lambda j: (0, 0)),
            pl.BlockSpec((8, DPS), lambda j: (0, 0)),
            pl.BlockSpec((B, BS, D), lambda j: (0, j, 0)),
        ],
        out_specs=pl.BlockSpec((B, BS, D), lambda j: (0, j, 0)),
        out_shape=jax.ShapeDtypeStruct((B, S, D), x.dtype),
        compiler_params=pltpu.CompilerParams(
            dimension_semantics=("parallel",),
        ),
    )(pf, pb, pp, fp, x)


# speedup vs baseline: 1.3716x; 1.0041x over previous
import functools
import math

import jax
import jax.numpy as jnp
from jax import lax
from jax.experimental import pallas as pl
from jax.experimental.pallas import tpu as pltpu
from jax.experimental.pallas import tpu_sc as plsc

D_MODEL = 1024
FPB = 43
BPB = 4
BPP = 4
DPS = D_MODEL // 4
PERIOD = FPB * BPB * BPP
PAT = 768
BS = PERIOD

_info = plsc.get_sparse_core_info()
NW = _info.num_cores * _info.num_subcores
RPW = PAT // NW


@functools.partial(
    pl.kernel,
    mesh=plsc.VectorSubcoreMesh(core_axis_name="c", subcore_axis_name="s"),
    out_type=jax.ShapeDtypeStruct((3 * PAT, DPS), jnp.float32),
    scratch_types=[
        pltpu.VMEM((3 * RPW,), jnp.int32),
        pltpu.VMEM((3 * RPW, DPS), jnp.float32),
        pltpu.SemaphoreType.DMA,
        pltpu.SemaphoreType.DMA,
    ],
)
def _gather_pattern(tab_hbm, idx_hbm, out_hbm, idx_v, rows_v, gsem, wsem):
    wid = lax.axis_index("s") * _info.num_cores + lax.axis_index("c")
    base = wid * (3 * RPW)
    pltpu.sync_copy(idx_hbm.at[wid], idx_v)
    pltpu.async_copy(tab_hbm.at[idx_v], rows_v, gsem).wait()
    pltpu.async_copy(rows_v, out_hbm.at[pl.ds(base, 3 * RPW)], wsem).wait()


def _add_pe_kernel(pf_ref, pb_ref, pp_ref, fp_ref, x_ref, o_ref):
    j = pl.program_id(0)
    row = j * BS + jax.lax.broadcasted_iota(jnp.int32, (BS, 1), 0)
    freq = fp_ref[0:1, :]
    phase = fp_ref[1:2, :]
    abs_pe = jnp.sin(row.astype(jnp.float32) * freq + phase)
    enc = jnp.concatenate(
        [pf_ref[:BS], pb_ref[:BS], pp_ref[:BS], abs_pe], axis=-1)
    o_ref[...] = x_ref[...] + enc[None, :, :]


def kernel(x, frame_embed, beat_embed, bar_embed, pe):
    B, S, D = x.shape
    table = jnp.concatenate([frame_embed, beat_embed, bar_embed], axis=0)
    r = jnp.arange(PAT, dtype=jnp.int32)
    idx_f = r % FPB
    idx_b = FPB + (r // FPB) % BPB
    idx_p = FPB + BPB + (r // (FPB * BPB)) % BPP
    idx_all = jnp.concatenate([idx_f, idx_b, idx_p]).reshape(NW, 3 * RPW)
    pat = _gather_pattern(table, idx_all)
    lane = jnp.arange(DPS)
    freq = jnp.exp((lane // 2 * 2).astype(jnp.float32) * (-math.log(10000.0) / DPS))
    phase = jnp.where(lane % 2 == 1, jnp.float32(math.pi / 2), jnp.float32(0.0))
    fp = jnp.zeros((8, DPS), x.dtype).at[0].set(freq).at[1].set(phase)
    return pl.pallas_call(
        _add_pe_kernel,
        grid=(pl.cdiv(S, BS),),
        in_specs=[
            pl.BlockSpec((PAT, DPS), lambda j: (0, 0)),
            pl.BlockSpec((PAT, DPS), lambda j: (1, 0)),
            pl.BlockSpec((PAT, DPS), lambda j: (2, 0)),
            pl.BlockSpec((8, DPS), lambda j: (0, 0)),
            pl.BlockSpec((B, BS, D), lambda j: (0, j, 0)),
        ],
        out_specs=pl.BlockSpec((B, BS, D), lambda j: (0, j, 0)),
        out_shape=jax.ShapeDtypeStruct((B, S, D), x.dtype),
        compiler_params=pltpu.CompilerParams(
            dimension_semantics=("parallel",),
        ),
    )(pat, pat, pat, fp, x)


# speedup vs baseline: 1.4516x; 1.0584x over previous
import functools
import math

import jax
import jax.numpy as jnp
from jax import lax
from jax.experimental import pallas as pl
from jax.experimental.pallas import tpu as pltpu
from jax.experimental.pallas import tpu_sc as plsc

D_MODEL = 1024
FPB = 43
BPB = 4
BPP = 4
DPS = D_MODEL // 4
PERIOD = FPB * BPB * BPP
PAT = 768
BS = PERIOD

_info = plsc.get_sparse_core_info()
NW = _info.num_cores * _info.num_subcores
RPW = PAT // NW


@functools.partial(
    pl.kernel,
    mesh=plsc.VectorSubcoreMesh(core_axis_name="c", subcore_axis_name="s"),
    out_type=jax.ShapeDtypeStruct((3 * PAT, DPS), jnp.float32),
    scratch_types=[
        pltpu.VMEM((FPB + BPB + BPP, DPS), jnp.float32),
        pltpu.VMEM((3 * RPW, DPS), jnp.float32),
        pltpu.SemaphoreType.DMA,
    ],
)
def _gather_pattern(tab_hbm, out_hbm, tab_v, rows_v, wsem):
    wid = lax.axis_index("s") * _info.num_cores + lax.axis_index("c")
    base = wid * (3 * RPW)
    pltpu.sync_copy(tab_hbm, tab_v)

    def _copy_row(q, carry):
        sq = base + q
        part = sq // PAT
        r = sq % PAT
        rowf = r % FPB
        rowb = FPB + (r // FPB) % BPB
        rowp = FPB + BPB + (r // (FPB * BPB)) % BPP
        row = jnp.where(part == 0, rowf, jnp.where(part == 1, rowb, rowp))
        for c in range(DPS // 16):
            rows_v[q, pl.ds(c * 16, 16)] = tab_v[row, pl.ds(c * 16, 16)]
        return carry

    lax.fori_loop(0, 3 * RPW, _copy_row, 0)
    pltpu.async_copy(rows_v, out_hbm.at[pl.ds(base, 3 * RPW)], wsem).wait()


def _add_pe_kernel(pf_ref, pb_ref, pp_ref, fp_ref, x_ref, o_ref):
    j = pl.program_id(0)
    row = j * BS + jax.lax.broadcasted_iota(jnp.int32, (BS, 1), 0)
    freq = fp_ref[0:1, :]
    phase = fp_ref[1:2, :]
    abs_pe = jnp.sin(row.astype(jnp.float32) * freq + phase)
    enc = jnp.concatenate(
        [pf_ref[:BS], pb_ref[:BS], pp_ref[:BS], abs_pe], axis=-1)
    o_ref[...] = x_ref[...] + enc[None, :, :]


def kernel(x, frame_embed, beat_embed, bar_embed, pe):
    B, S, D = x.shape
    table = jnp.concatenate([frame_embed, beat_embed, bar_embed], axis=0)
    pat = _gather_pattern(table)
    lane = jnp.arange(DPS)
    freq = jnp.exp((lane // 2 * 2).astype(jnp.float32) * (-math.log(10000.0) / DPS))
    phase = jnp.where(lane % 2 == 1, jnp.float32(math.pi / 2), jnp.float32(0.0))
    fp = jnp.zeros((8, DPS), x.dtype).at[0].set(freq).at[1].set(phase)
    return pl.pallas_call(
        _add_pe_kernel,
        grid=(pl.cdiv(S, BS),),
        in_specs=[
            pl.BlockSpec((PAT, DPS), lambda j: (0, 0)),
            pl.BlockSpec((PAT, DPS), lambda j: (1, 0)),
            pl.BlockSpec((PAT, DPS), lambda j: (2, 0)),
            pl.BlockSpec((8, DPS), lambda j: (0, 0)),
            pl.BlockSpec((B, BS, D), lambda j: (0, j, 0)),
        ],
        out_specs=pl.BlockSpec((B, BS, D), lambda j: (0, j, 0)),
        out_shape=jax.ShapeDtypeStruct((B, S, D), x.dtype),
        compiler_params=pltpu.CompilerParams(
            dimension_semantics=("parallel",),
        ),
    )(pat, pat, pat, fp, x)
